# baseline (speedup 1.0000x reference)
import jax
import jax.numpy as jnp
from jax.experimental import pallas as pl
from jax.experimental.pallas import tpu as pltpu

_NG = 256


def _pool_body(batch_ref, x_ref, W1_ref, b1_ref, W2_ref, b2_ref, out_ref,
               accx, accw, accc, m_ref, *, n_total):
    i = pl.program_id(0)
    nb = pl.num_programs(0)

    @pl.when(i == 0)
    def _init():
        accx[...] = jnp.zeros_like(accx)
        accw[...] = jnp.zeros_like(accw)
        accc[...] = jnp.zeros_like(accc)
        m_ref[0, 0] = -jnp.inf

    x = x_ref[...]
    h = jnp.dot(x, W1_ref[...], preferred_element_type=jnp.float32)
    h = jnp.maximum(h + b1_ref[...], 0.0)
    wf = jnp.dot(h, W2_ref[...], preferred_element_type=jnp.float32)
    wf = wf + b2_ref[0, 0]

    bm = jnp.max(wf)
    m_old = m_ref[0, 0]
    m_new = jnp.maximum(m_old, bm)
    scale = jnp.exp(m_old - m_new)
    m_ref[0, 0] = m_new

    ew = jnp.exp(wf - m_new)
    ids = batch_ref[0]
    gids = jax.lax.broadcasted_iota(jnp.int32, (_NG, ids.shape[1]), 0)
    oh = (gids == ids).astype(jnp.float32)

    accx[...] = accx[...] * scale + jnp.dot(oh, ew * x,
                                            preferred_element_type=jnp.float32)
    accw[...] = accw[...] * scale + jnp.dot(oh, ew,
                                            preferred_element_type=jnp.float32)
    accc[...] = accc[...] + jnp.sum(oh, axis=1, keepdims=True)

    @pl.when(i == nb - 1)
    def _fin():
        cnt = jnp.maximum(accc[...], 1.0)
        sw = accw[...]
        denom = (sw / cnt) * float(n_total) + 1e-8
        out_ref[...] = accx[...] / (denom * cnt)


def _build_call(N, D, H, B, interpret=False):
    nb = N // B
    import functools
    body = functools.partial(_pool_body, n_total=N)
    return pl.pallas_call(
        body,
        grid=(nb,),
        in_specs=[
            pl.BlockSpec((1, 1, B), lambda i: (i, 0, 0)),
            pl.BlockSpec((B, D), lambda i: (i, 0)),
            pl.BlockSpec((D, H), lambda i: (0, 0)),
            pl.BlockSpec((1, H), lambda i: (0, 0)),
            pl.BlockSpec((H, 1), lambda i: (0, 0)),
            pl.BlockSpec(memory_space=pltpu.SMEM),
        ],
        out_specs=pl.BlockSpec((_NG, D), lambda i: (0, 0)),
        out_shape=jax.ShapeDtypeStruct((_NG, D), jnp.float32),
        scratch_shapes=[
            pltpu.VMEM((_NG, D), jnp.float32),
            pltpu.VMEM((_NG, 1), jnp.float32),
            pltpu.VMEM((_NG, 1), jnp.float32),
            pltpu.SMEM((1, 1), jnp.float32),
        ],
        interpret=interpret,
    )


def kernel(x, batch, W1, b1, W2, b2):
    N, D = x.shape
    H = W1.shape[1]
    B = 2000
    nb = N // B
    batch3d = batch.astype(jnp.int32).reshape(nb, 1, B)
    b1r = b1.reshape(1, H)
    b2r = b2.reshape(1, 1)
    call = _build_call(N, D, H, B)
    return call(batch3d, x, W1, b1r, W2, b2r)


# speedup vs baseline: 1.0821x; 1.0821x over previous
import jax
import jax.numpy as jnp
from jax.experimental import pallas as pl
from jax.experimental.pallas import tpu as pltpu

_NG = 256


def _pool_body(batch_ref, x_ref, W1_ref, b1_ref, W2_ref, b2_ref, out_ref,
               accx, accw, accc, m_ref, *, n_total):
    i = pl.program_id(0)
    nb = pl.num_programs(0)

    @pl.when(i == 0)
    def _init():
        accx[...] = jnp.zeros_like(accx)
        accw[...] = jnp.zeros_like(accw)
        accc[...] = jnp.zeros_like(accc)
        m_ref[0, 0] = -jnp.inf

    x = x_ref[...]
    h = jnp.dot(x, W1_ref[...], preferred_element_type=jnp.float32)
    h = jnp.maximum(h + b1_ref[...], 0.0)
    wf = jnp.dot(h, W2_ref[...], preferred_element_type=jnp.float32)
    wf = wf + b2_ref[0, 0]

    bm = jnp.max(wf)
    m_old = m_ref[0, 0]
    m_new = jnp.maximum(m_old, bm)
    scale = jnp.exp(m_old - m_new)
    m_ref[0, 0] = m_new

    ew = jnp.exp(wf - m_new)
    ids = batch_ref[0]
    gids = jax.lax.broadcasted_iota(jnp.int32, (_NG, ids.shape[1]), 0)
    oh = (gids == ids).astype(jnp.float32)

    accx[...] = accx[...] * scale + jnp.dot(oh, ew * x,
                                            preferred_element_type=jnp.float32)
    accw[...] = accw[...] * scale + jnp.dot(oh, ew,
                                            preferred_element_type=jnp.float32)
    accc[...] = accc[...] + jnp.sum(oh, axis=1, keepdims=True)

    @pl.when(i == nb - 1)
    def _fin():
        cnt = jnp.maximum(accc[...], 1.0)
        sw = accw[...]
        denom = (sw / cnt) * float(n_total) + 1e-8
        out_ref[...] = accx[...] / (denom * cnt)


def _build_call(N, D, H, B, interpret=False):
    nb = N // B
    import functools
    body = functools.partial(_pool_body, n_total=N)
    return pl.pallas_call(
        body,
        grid=(nb,),
        in_specs=[
            pl.BlockSpec((1, 1, B), lambda i: (i, 0, 0)),
            pl.BlockSpec((B, D), lambda i: (i, 0)),
            pl.BlockSpec((D, H), lambda i: (0, 0)),
            pl.BlockSpec((1, H), lambda i: (0, 0)),
            pl.BlockSpec((H, 1), lambda i: (0, 0)),
            pl.BlockSpec(memory_space=pltpu.SMEM),
        ],
        out_specs=pl.BlockSpec((_NG, D), lambda i: (0, 0)),
        out_shape=jax.ShapeDtypeStruct((_NG, D), jnp.float32),
        scratch_shapes=[
            pltpu.VMEM((_NG, D), jnp.float32),
            pltpu.VMEM((_NG, 1), jnp.float32),
            pltpu.VMEM((_NG, 1), jnp.float32),
            pltpu.SMEM((1, 1), jnp.float32),
        ],
        interpret=interpret,
    )


def kernel(x, batch, W1, b1, W2, b2):
    N, D = x.shape
    H = W1.shape[1]
    B = 5000
    nb = N // B
    batch3d = batch.astype(jnp.int32).reshape(nb, 1, B)
    b1r = b1.reshape(1, H)
    b2r = b2.reshape(1, 1)
    call = _build_call(N, D, H, B)
    return call(batch3d, x, W1, b1r, W2, b2r)


# speedup vs baseline: 1.4737x; 1.3619x over previous
import jax
import jax.numpy as jnp
from jax.experimental import pallas as pl
from jax.experimental.pallas import tpu as pltpu

_NG = 256


def _pool_body(batch_ref, x_ref, W1_ref, b1_ref, W2_ref, b2_ref, out_ref,
               accx, accw, accc, m_ref, *, n_total):
    i = pl.program_id(0)
    nb = pl.num_programs(0)

    @pl.when(i == 0)
    def _init():
        accx[...] = jnp.zeros_like(accx)
        accw[...] = jnp.zeros_like(accw)
        accc[...] = jnp.zeros_like(accc)
        m_ref[0, 0] = -jnp.inf

    x = x_ref[...]
    h = jnp.dot(x, W1_ref[...], preferred_element_type=jnp.float32)
    h = jnp.maximum(h + b1_ref[...], 0.0)
    wf = jnp.dot(h, W2_ref[...], preferred_element_type=jnp.float32)
    wf = wf + b2_ref[0, 0]

    bm = jnp.max(wf)
    m_old = m_ref[0, 0]
    m_new = jnp.maximum(m_old, bm)
    scale = jnp.exp(m_old - m_new)
    m_ref[0, 0] = m_new

    ew = jnp.exp(wf - m_new)
    ids = batch_ref[0]
    gids = jax.lax.broadcasted_iota(jnp.int32, (_NG, ids.shape[1]), 0)
    oh = (gids == ids).astype(jnp.float32)

    accx[...] = accx[...] * scale + jnp.dot(oh, ew * x,
                                            preferred_element_type=jnp.float32)
    accw[...] = accw[...] * scale + jnp.dot(oh, ew,
                                            preferred_element_type=jnp.float32)
    accc[...] = accc[...] + jnp.sum(oh, axis=1, keepdims=True)

    @pl.when(i == nb - 1)
    def _fin():
        cnt = jnp.maximum(accc[...], 1.0)
        sw = accw[...]
        denom = (sw / cnt) * float(n_total) + 1e-8
        out_ref[...] = accx[...] / (denom * cnt)


def _build_call(N, D, H, B, interpret=False):
    nb = N // B
    import functools
    body = functools.partial(_pool_body, n_total=N)
    return pl.pallas_call(
        body,
        grid=(nb,),
        in_specs=[
            pl.BlockSpec((1, 1, B), lambda i: (i, 0, 0)),
            pl.BlockSpec((B, D), lambda i: (i, 0)),
            pl.BlockSpec((D, H), lambda i: (0, 0)),
            pl.BlockSpec((1, H), lambda i: (0, 0)),
            pl.BlockSpec((H, 1), lambda i: (0, 0)),
            pl.BlockSpec(memory_space=pltpu.SMEM),
        ],
        out_specs=pl.BlockSpec((_NG, D), lambda i: (0, 0)),
        out_shape=jax.ShapeDtypeStruct((_NG, D), jnp.float32),
        scratch_shapes=[
            pltpu.VMEM((_NG, D), jnp.float32),
            pltpu.VMEM((_NG, 1), jnp.float32),
            pltpu.VMEM((_NG, 1), jnp.float32),
            pltpu.SMEM((1, 1), jnp.float32),
        ],
        interpret=interpret,
    )


def kernel(x, batch, W1, b1, W2, b2):
    N, D = x.shape
    H = W1.shape[1]
    B = 10000
    nb = N // B
    batch3d = batch.astype(jnp.int32).reshape(nb, 1, B)
    b1r = b1.reshape(1, H)
    b2r = b2.reshape(1, 1)
    call = _build_call(N, D, H, B)
    return call(batch3d, x, W1, b1r, W2, b2r)


# speedup vs baseline: 1.5271x; 1.0363x over previous
import jax
import jax.numpy as jnp
from jax.experimental import pallas as pl
from jax.experimental.pallas import tpu as pltpu

_NG = 256


def _pool_body(batch_ref, x_ref, W1_ref, b1_ref, W2_ref, b2_ref, out_ref,
               accx, accw, accc, m_ref, *, n_total):
    i = pl.program_id(0)
    nb = pl.num_programs(0)

    @pl.when(i == 0)
    def _init():
        accx[...] = jnp.zeros_like(accx)
        accw[...] = jnp.zeros_like(accw)
        accc[...] = jnp.zeros_like(accc)
        m_ref[0, 0] = -jnp.inf

    x = x_ref[...]
    h = jnp.dot(x, W1_ref[...], preferred_element_type=jnp.float32)
    h = jnp.maximum(h + b1_ref[...], 0.0)
    wf = jnp.dot(h, W2_ref[...], preferred_element_type=jnp.float32)
    wf = wf + b2_ref[0, 0]

    bm = jnp.max(wf)
    m_old = m_ref[0, 0]
    m_new = jnp.maximum(m_old, bm)
    scale = jnp.exp(m_old - m_new)
    m_ref[0, 0] = m_new

    ew = jnp.exp(wf - m_new)
    ids = batch_ref[0]
    gids = jax.lax.broadcasted_iota(jnp.int32, (_NG, ids.shape[1]), 0)
    oh = (gids == ids).astype(jnp.float32)

    accx[...] = accx[...] * scale + jnp.dot(oh, ew * x,
                                            preferred_element_type=jnp.float32)
    accw[...] = accw[...] * scale + jnp.dot(oh, ew,
                                            preferred_element_type=jnp.float32)
    accc[...] = accc[...] + jnp.sum(oh, axis=1, keepdims=True)

    @pl.when(i == nb - 1)
    def _fin():
        cnt = jnp.maximum(accc[...], 1.0)
        sw = accw[...]
        denom = (sw / cnt) * float(n_total) + 1e-8
        out_ref[...] = accx[...] / (denom * cnt)


def _build_call(N, D, H, B, interpret=False):
    nb = N // B
    import functools
    body = functools.partial(_pool_body, n_total=N)
    return pl.pallas_call(
        body,
        grid=(nb,),
        in_specs=[
            pl.BlockSpec((1, 1, B), lambda i: (i, 0, 0)),
            pl.BlockSpec((B, D), lambda i: (i, 0)),
            pl.BlockSpec((D, H), lambda i: (0, 0)),
            pl.BlockSpec((1, H), lambda i: (0, 0)),
            pl.BlockSpec((H, 1), lambda i: (0, 0)),
            pl.BlockSpec(memory_space=pltpu.SMEM),
        ],
        out_specs=pl.BlockSpec((_NG, D), lambda i: (0, 0)),
        out_shape=jax.ShapeDtypeStruct((_NG, D), jnp.float32),
        scratch_shapes=[
            pltpu.VMEM((_NG, D), jnp.float32),
            pltpu.VMEM((_NG, 1), jnp.float32),
            pltpu.VMEM((_NG, 1), jnp.float32),
            pltpu.SMEM((1, 1), jnp.float32),
        ],
        interpret=interpret,
    )


def kernel(x, batch, W1, b1, W2, b2):
    N, D = x.shape
    H = W1.shape[1]
    B = 20000
    nb = N // B
    batch3d = batch.astype(jnp.int32).reshape(nb, 1, B)
    b1r = b1.reshape(1, H)
    b2r = b2.reshape(1, 1)
    call = _build_call(N, D, H, B)
    return call(batch3d, x, W1, b1r, W2, b2r)
